# baseline (speedup 1.0000x reference)
import functools
import math

import jax
import jax.numpy as jnp
import numpy as np
from jax import lax
from jax.experimental import pallas as pl
from jax.experimental.pallas import tpu as pltpu
from jax.experimental.pallas import tpu_sc as plsc

_TOPK = 16
_NW = 144
_NS = 576
_C = 768
_NT = 144
_B = 32
_LANES = 16
_NCHUNK = _C // _LANES
_BSTEP = 4


def _avg_matrix():
    A = np.zeros((_NW, _NS), np.float32)
    for w in range(_NW):
        t0 = (w // 12) * 48 + (w % 12) * 2
        for off in (0, 1, 24, 25):
            A[w, t0 + off] = 0.25
    return A

_A_CONST = _avg_matrix()


def _tc_kernel(gab_ref, z_ref, x_ref, a_ref, tok_ref, mf_ref, scores_ref):
    f32 = jnp.float32
    b = pl.program_id(0)

    @pl.when(b == 0)
    def _():
        o2 = lax.broadcasted_iota(jnp.int32, (16, 32), 0)
        col = lax.broadcasted_iota(jnp.int32, (16, 32), 1)
        i = col // 16
        p = o2 // 4
        q = o2 % 4
        pf = p.astype(f32)
        qf = q.astype(f32)
        ypos = pf * (2.0 / 3.0) - 1.0
        xpos = qf * (2.0 / 3.0) - 1.0
        smap = jnp.zeros((16, 32), f32)
        for gi in range(16):
            th = gab_ref[0, gi]
            sg = gab_ref[1, gi]
            lm = gab_ref[2, gi]
            ps = gab_ref[3, gi]
            gm = gab_ref[4, gi]
            am = gab_ref[5, gi]
            thv = th + jnp.zeros((16, 32), f32)
            ct = jnp.cos(thv)
            st = jnp.sin(thv)
            xr = xpos * ct + ypos * st
            yr = -xpos * st + ypos * ct
            sig = abs(sg) + 0.5
            lmv = abs(lm) + 0.5
            gmv = abs(gm) + 0.5
            gv = jnp.exp(-(xr * xr + (gmv * yr) ** 2) / (2.0 * sig * sig)) \
                * jnp.cos((2.0 * math.pi) * xr / lmv + ps)
            smap = smap + am * gv
        wa = jnp.where(p == 0, 1.0,
                       jnp.where(p == 1, 0.75,
                                 jnp.where(p == 2, 0.25, 0.0))).astype(f32)
        wv = jnp.where(i == 0, wa, 1.0 - wa)
        mf_ref[...] = (1.0 + smap) * wv

    for bi in range(_BSTEP):
        zb = z_ref[bi]
        xb = x_ref[bi]
        zmax = jnp.max(zb, axis=0, keepdims=True)
        resp = lax.dot_general(zmax.astype(jnp.bfloat16),
                               xb.astype(jnp.bfloat16),
                               (((1,), (1,)), ((), ())),
                               preferred_element_type=f32)
        sc = lax.dot_general(resp, a_ref[...], (((1,), (1,)), ((), ())),
                             precision=lax.Precision.HIGHEST,
                             preferred_element_type=f32)
        scores_ref[pl.ds(b * _BSTEP + bi, 1), :] = sc

    @pl.when(b == _B // _BSTEP - 1)
    def _():
        scs = scores_ref[...]
        iota_w = lax.broadcasted_iota(jnp.int32, (_B, _NW), 1)
        big = jnp.int32(1 << 30)
        colio = lax.broadcasted_iota(jnp.int32, (_B, 4 * _TOPK), 1)
        brow = lax.broadcasted_iota(jnp.int32, (_B, 4 * _TOPK), 0)
        m4 = colio % 4
        offc = jnp.where(m4 == 0, 0,
                         jnp.where(m4 == 1, 1,
                                   jnp.where(m4 == 2, 24, 25)))
        tokrow = offc + brow * _NS
        for k in range(_TOPK):
            mx = jnp.max(scs, axis=1, keepdims=True)
            idx = jnp.min(jnp.where(scs == mx, iota_w, big), axis=1,
                          keepdims=True)
            scs = jnp.where(iota_w == idx, -jnp.inf, scs)
            t00 = (idx // 12) * 48 + (idx % 12) * 2
            tokrow = tokrow + jnp.where(colio // 4 == k, t00, 0)
        tok_ref[...] = tokrow


def _tc_call(gab, z, x):
    return pl.pallas_call(
        _tc_kernel,
        grid=(_B // _BSTEP,),
        in_specs=[
            pl.BlockSpec(memory_space=pltpu.SMEM),
            pl.BlockSpec((_BSTEP, _NT, _C), lambda b: (b, 0, 0)),
            pl.BlockSpec((_BSTEP, _NS, _C), lambda b: (b, 0, 0)),
            pl.BlockSpec((_NW, _NS), lambda b: (0, 0)),
        ],
        out_specs=[
            pl.BlockSpec((_B, 4 * _TOPK), lambda b: (0, 0)),
            pl.BlockSpec((16, 32), lambda b: (0, 0)),
        ],
        out_shape=[
            jax.ShapeDtypeStruct((_B, 4 * _TOPK), jnp.int32),
            jax.ShapeDtypeStruct((16, 32), jnp.float32),
        ],
        scratch_shapes=[pltpu.VMEM((_B, _NW), jnp.float32)],
    )(gab, z, x, jnp.asarray(_A_CONST))


def _sc_kernel(x_hbm, tok_hbm, mf_hbm, out_hbm,
               tok_v, mf_v, g_v, o_v0, o_v1, sem, sem0, sem1):
    wid = lax.axis_index("s") * 2 + lax.axis_index("c")
    pltpu.sync_copy(tok_hbm.at[wid], tok_v)
    pltpu.sync_copy(mf_hbm, mf_v)
    pltpu.async_copy(x_hbm.at[tok_v], g_v, sem).wait()

    w0 = [mf_v[o, pl.ds(0, _LANES)] for o in range(16)]
    w1 = [mf_v[o, pl.ds(_LANES, _LANES)] for o in range(16)]

    bufs = (o_v0, o_v1)
    sems = (sem0, sem1)
    pending = [None, None]
    for k in range(_TOPK):
        o_v = bufs[k % 2]
        if pending[k % 2] is not None:
            pending[k % 2].wait()

        def body(v, carry):
            g00 = g_v[4 * k + 0, pl.ds(v * _LANES, _LANES)]
            g01 = g_v[4 * k + 1, pl.ds(v * _LANES, _LANES)]
            g10 = g_v[4 * k + 2, pl.ds(v * _LANES, _LANES)]
            g11 = g_v[4 * k + 3, pl.ds(v * _LANES, _LANES)]
            t0 = (g00, 0.75 * g00 + 0.25 * g01, 0.25 * g00 + 0.75 * g01, g01)
            t1 = (g10, 0.75 * g10 + 0.25 * g11, 0.25 * g10 + 0.75 * g11, g11)
            for o in range(16):
                q = o % 4
                o_v[o, pl.ds(v * _LANES, _LANES)] = w0[o] * t0[q] + w1[o] * t1[q]
            return carry
        lax.fori_loop(0, _NCHUNK, body, 0)
        pending[k % 2] = pltpu.async_copy(
            o_v, out_hbm.at[wid, pl.ds(16 * k, 16)], sems[k % 2])
    pending[0].wait()
    pending[1].wait()


@jax.jit
def kernel(z, x, gabor_theta, gabor_sigma, gabor_lambda, gabor_psi,
           gabor_gamma, gabor_amp):
    gab = jnp.stack([gabor_theta, gabor_sigma, gabor_lambda, gabor_psi,
                     gabor_gamma, gabor_amp], axis=0)
    tok, mf = _tc_call(gab, z, x)
    xflat = x.reshape(_B * _NS, _C)

    sck = functools.partial(
        pl.kernel,
        mesh=plsc.VectorSubcoreMesh(core_axis_name="c", subcore_axis_name="s"),
        out_type=jax.ShapeDtypeStruct((_B, _TOPK * 16, _C), jnp.float32),
        scratch_types=[
            pltpu.VMEM((4 * _TOPK,), jnp.int32),
            pltpu.VMEM((16, 32), jnp.float32),
            pltpu.VMEM((4 * _TOPK, _C), jnp.float32),
            pltpu.VMEM((16, _C), jnp.float32),
            pltpu.VMEM((16, _C), jnp.float32),
            pltpu.SemaphoreType.DMA,
            pltpu.SemaphoreType.DMA,
            pltpu.SemaphoreType.DMA,
        ],
    )(_sc_kernel)
    return sck(xflat, tok, mf)


# speedup vs baseline: 1.0164x; 1.0164x over previous
import functools
import math

import jax
import jax.numpy as jnp
import numpy as np
from jax import lax
from jax.experimental import pallas as pl
from jax.experimental.pallas import tpu as pltpu
from jax.experimental.pallas import tpu_sc as plsc

_TOPK = 16
_NW = 144
_NS = 576
_C = 768
_NT = 144
_B = 32
_LANES = 16
_NCHUNK = _C // _LANES
_BSTEP = 8


def _avg_matrix():
    A = np.zeros((_NW, _NS), np.float32)
    for w in range(_NW):
        t0 = (w // 12) * 48 + (w % 12) * 2
        for off in (0, 1, 24, 25):
            A[w, t0 + off] = 0.25
    return A

_A_CONST = _avg_matrix()


def _tc_kernel(gab_ref, z_ref, x_ref, a_ref, tok_ref, mf_ref, scores_ref):
    f32 = jnp.float32
    b = pl.program_id(0)

    @pl.when(b == 0)
    def _():
        o2 = lax.broadcasted_iota(jnp.int32, (16, 32), 0)
        col = lax.broadcasted_iota(jnp.int32, (16, 32), 1)
        i = col // 16
        p = o2 // 4
        q = o2 % 4
        pf = p.astype(f32)
        qf = q.astype(f32)
        ypos = pf * (2.0 / 3.0) - 1.0
        xpos = qf * (2.0 / 3.0) - 1.0
        smap = jnp.zeros((16, 32), f32)
        for gi in range(16):
            th = gab_ref[0, gi]
            sg = gab_ref[1, gi]
            lm = gab_ref[2, gi]
            ps = gab_ref[3, gi]
            gm = gab_ref[4, gi]
            am = gab_ref[5, gi]
            thv = th + jnp.zeros((16, 32), f32)
            ct = jnp.cos(thv)
            st = jnp.sin(thv)
            xr = xpos * ct + ypos * st
            yr = -xpos * st + ypos * ct
            sig = abs(sg) + 0.5
            lmv = abs(lm) + 0.5
            gmv = abs(gm) + 0.5
            gv = jnp.exp(-(xr * xr + (gmv * yr) ** 2) / (2.0 * sig * sig)) \
                * jnp.cos((2.0 * math.pi) * xr / lmv + ps)
            smap = smap + am * gv
        wa = jnp.where(p == 0, 1.0,
                       jnp.where(p == 1, 0.75,
                                 jnp.where(p == 2, 0.25, 0.0))).astype(f32)
        wv = jnp.where(i == 0, wa, 1.0 - wa)
        mf_ref[...] = (1.0 + smap) * wv

    for bi in range(_BSTEP):
        zb = z_ref[bi]
        xb = x_ref[bi]
        zmax = jnp.max(zb, axis=0, keepdims=True)
        resp = lax.dot_general(zmax.astype(jnp.bfloat16),
                               xb.astype(jnp.bfloat16),
                               (((1,), (1,)), ((), ())),
                               preferred_element_type=f32)
        sc = lax.dot_general(resp, a_ref[...], (((1,), (1,)), ((), ())),
                             precision=lax.Precision.HIGHEST,
                             preferred_element_type=f32)
        scores_ref[pl.ds(b * _BSTEP + bi, 1), :] = sc

    @pl.when(b == _B // _BSTEP - 1)
    def _():
        scs = scores_ref[...]
        iota_w = lax.broadcasted_iota(jnp.int32, (_B, _NW), 1)
        big = jnp.int32(1 << 30)
        colio = lax.broadcasted_iota(jnp.int32, (_B, 4 * _TOPK), 1)
        brow = lax.broadcasted_iota(jnp.int32, (_B, 4 * _TOPK), 0)
        m4 = colio % 4
        offc = jnp.where(m4 == 0, 0,
                         jnp.where(m4 == 1, 1,
                                   jnp.where(m4 == 2, 24, 25)))
        tokrow = offc + brow * _NS
        for k in range(_TOPK):
            mx = jnp.max(scs, axis=1, keepdims=True)
            idx = jnp.min(jnp.where(scs == mx, iota_w, big), axis=1,
                          keepdims=True)
            scs = jnp.where(iota_w == idx, -jnp.inf, scs)
            t00 = (idx // 12) * 48 + (idx % 12) * 2
            tokrow = tokrow + jnp.where(colio // 4 == k, t00, 0)
        tok_ref[...] = tokrow


def _tc_call(gab, z, x):
    return pl.pallas_call(
        _tc_kernel,
        grid=(_B // _BSTEP,),
        in_specs=[
            pl.BlockSpec(memory_space=pltpu.SMEM),
            pl.BlockSpec((_BSTEP, _NT, _C), lambda b: (b, 0, 0)),
            pl.BlockSpec((_BSTEP, _NS, _C), lambda b: (b, 0, 0)),
            pl.BlockSpec((_NW, _NS), lambda b: (0, 0)),
        ],
        out_specs=[
            pl.BlockSpec((_B, 4 * _TOPK), lambda b: (0, 0)),
            pl.BlockSpec((16, 32), lambda b: (0, 0)),
        ],
        out_shape=[
            jax.ShapeDtypeStruct((_B, 4 * _TOPK), jnp.int32),
            jax.ShapeDtypeStruct((16, 32), jnp.float32),
        ],
        scratch_shapes=[pltpu.VMEM((_B, _NW), jnp.float32)],
    )(gab, z, x, jnp.asarray(_A_CONST))


def _sc_kernel(x_hbm, tok_hbm, mf_hbm, out_hbm,
               tok_v, mf_v, g_v, o_v0, o_v1,
               gsem0, gsem1, gsem2, gsem3, sem0, sem1):
    gsems = (gsem0, gsem1, gsem2, gsem3)
    wid = lax.axis_index("s") * 2 + lax.axis_index("c")
    pltpu.sync_copy(tok_hbm.at[wid], tok_v)
    gathers = [
        pltpu.async_copy(x_hbm.at[tok_v.at[pl.ds(16 * i, 16)]],
                         g_v.at[pl.ds(16 * i, 16)], gsems[i])
        for i in range(4)
    ]
    pltpu.sync_copy(mf_hbm, mf_v)

    w0 = [mf_v[o, pl.ds(0, _LANES)] for o in range(16)]
    w1 = [mf_v[o, pl.ds(_LANES, _LANES)] for o in range(16)]

    bufs = (o_v0, o_v1)
    sems = (sem0, sem1)
    pending = [None, None]
    for k in range(_TOPK):
        if k % 4 == 0:
            gathers[k // 4].wait()
        o_v = bufs[k % 2]
        if pending[k % 2] is not None:
            pending[k % 2].wait()

        def body(v, carry):
            g00 = g_v[4 * k + 0, pl.ds(v * _LANES, _LANES)]
            g01 = g_v[4 * k + 1, pl.ds(v * _LANES, _LANES)]
            g10 = g_v[4 * k + 2, pl.ds(v * _LANES, _LANES)]
            g11 = g_v[4 * k + 3, pl.ds(v * _LANES, _LANES)]
            t0 = (g00, 0.75 * g00 + 0.25 * g01, 0.25 * g00 + 0.75 * g01, g01)
            t1 = (g10, 0.75 * g10 + 0.25 * g11, 0.25 * g10 + 0.75 * g11, g11)
            for o in range(16):
                q = o % 4
                o_v[o, pl.ds(v * _LANES, _LANES)] = w0[o] * t0[q] + w1[o] * t1[q]
            return carry
        lax.fori_loop(0, _NCHUNK, body, 0)
        pending[k % 2] = pltpu.async_copy(
            o_v, out_hbm.at[wid, pl.ds(16 * k, 16)], sems[k % 2])
    pending[0].wait()
    pending[1].wait()


@jax.jit
def kernel(z, x, gabor_theta, gabor_sigma, gabor_lambda, gabor_psi,
           gabor_gamma, gabor_amp):
    gab = jnp.stack([gabor_theta, gabor_sigma, gabor_lambda, gabor_psi,
                     gabor_gamma, gabor_amp], axis=0)
    tok, mf = _tc_call(gab, z, x)
    xflat = x.reshape(_B * _NS, _C)

    sck = functools.partial(
        pl.kernel,
        mesh=plsc.VectorSubcoreMesh(core_axis_name="c", subcore_axis_name="s"),
        out_type=jax.ShapeDtypeStruct((_B, _TOPK * 16, _C), jnp.float32),
        scratch_types=[
            pltpu.VMEM((4 * _TOPK,), jnp.int32),
            pltpu.VMEM((16, 32), jnp.float32),
            pltpu.VMEM((4 * _TOPK, _C), jnp.float32),
            pltpu.VMEM((16, _C), jnp.float32),
            pltpu.VMEM((16, _C), jnp.float32),
            pltpu.SemaphoreType.DMA,
            pltpu.SemaphoreType.DMA,
            pltpu.SemaphoreType.DMA,
            pltpu.SemaphoreType.DMA,
            pltpu.SemaphoreType.DMA,
            pltpu.SemaphoreType.DMA,
        ],
    )(_sc_kernel)
    return sck(xflat, tok, mf)
